# Initial kernel scaffold; baseline (speedup 1.0000x reference)
import jax
import jax.numpy as jnp
from jax.experimental import pallas as pl


def kernel(x, edge_index, edge_attr, W_src, W_dst, a_src, a_dst, a_edge):
    raise NotImplementedError("write your pallas kernel here")


# speedup vs baseline: 36.9556x; 36.9556x over previous
import dataclasses
import functools

import jax
import jax.numpy as jnp
from jax import lax
from jax.experimental import pallas as pl
from jax.experimental.pallas import tpu as pltpu
from jax.experimental.pallas import tpu_sc as plsc

N = 10000
E = 320000
D = 128
DE = 4
H = 4
F = 32
HF = H * F

NC = 2
NS = 16
NW = NC * NS
L = 16

W_EDGES = 80
NWIN = E // W_EDGES
SUBW = NWIN // NW
TPAD = 4 * N + 960
TSLICE = TPAD // NS

_prec = lax.Precision.HIGHEST


def _sc_params():
    cp = pltpu.CompilerParams()
    if "needs_layout_passes" in pltpu.CompilerParams.__dataclass_fields__:
        cp = dataclasses.replace(cp, needs_layout_passes=False)
    return cp


def _mesh():
    return plsc.VectorSubcoreMesh(core_axis_name="c", subcore_axis_name="s")


def _node_proj(x, W_src, W_dst, aW):
    def body(x_ref, ws_ref, wd_ref, aw_ref, fs_ref, fd_ref, sa_ref):
        xx = x_ref[...]
        fs_ref[...] = jax.lax.dot(xx, ws_ref[...], precision=_prec)
        fd_ref[...] = jax.lax.dot(xx, wd_ref[...], precision=_prec)
        sa_ref[...] = jax.lax.dot(xx, aw_ref[...], precision=_prec)

    return pl.pallas_call(
        body,
        out_shape=[
            jax.ShapeDtypeStruct((N, HF), jnp.float32),
            jax.ShapeDtypeStruct((N, HF), jnp.float32),
            jax.ShapeDtypeStruct((N, 2 * H), jnp.float32),
        ],
    )(x, W_src, W_dst, aW)


def _inv_esum(esum_part):
    R = TPAD // 128

    def body(p_ref, o_ref):
        o_ref[...] = 1.0 / (p_ref[0] + p_ref[1])

    out = pl.pallas_call(
        body,
        out_shape=jax.ShapeDtypeStruct((R, 128), jnp.float32),
    )(esum_part.reshape(2, R, 128))
    return out.reshape(TPAD)


def _combine(out_part, feat_dst):
    BN = 1000

    def body(p_ref, fd_ref, o_ref):
        o_ref[...] = p_ref[0] + p_ref[1] + fd_ref[...]

    return pl.pallas_call(
        body,
        grid=(N // BN,),
        in_specs=[
            pl.BlockSpec((2, BN, HF), lambda i: (0, i, 0)),
            pl.BlockSpec((BN, HF), lambda i: (i, 0)),
        ],
        out_specs=pl.BlockSpec((BN, HF), lambda i: (i, 0)),
        out_shape=jax.ShapeDtypeStruct((N, HF), jnp.float32),
    )(out_part, feat_dst)


def _attn_kernel(asrc_flat, adst_flat, src, dst, ea_flat, ae_flat):
    @functools.partial(
        pl.kernel,
        mesh=_mesh(),
        out_type=[
            jax.ShapeDtypeStruct((4 * E,), jnp.float32),
            jax.ShapeDtypeStruct((NC, TPAD), jnp.float32),
        ],
        scratch_types=[
            pltpu.VMEM((4 * N,), jnp.float32),
            pltpu.VMEM((4 * N,), jnp.float32),
            pltpu.VMEM((W_EDGES,), jnp.int32),
            pltpu.VMEM((W_EDGES,), jnp.int32),
            pltpu.VMEM((4 * W_EDGES,), jnp.float32),
            pltpu.VMEM((4 * W_EDGES,), jnp.float32),
            pltpu.VMEM((2, 128), jnp.int32),
            pltpu.VMEM((64,), jnp.int32),
            pltpu.VMEM((16,), jnp.float32),
            pltpu.VMEM((TSLICE,), jnp.float32),
            pltpu.VMEM_SHARED((TPAD,), jnp.float32),
        ],
        compiler_params=_sc_params(),
    )
    def kern(asrc_hbm, adst_hbm, src_hbm, dst_hbm, ea_hbm, ae_hbm,
             w_hbm, part_hbm,
             asrc_t, adst_t, srcb, dstb, eab, wb, idxb2, idxb1, aetab,
             zb, esum_sh):
        cid = lax.axis_index("c")
        sid = lax.axis_index("s")
        wid = cid * NS + sid

        pltpu.sync_copy(asrc_hbm, asrc_t)
        pltpu.sync_copy(adst_hbm, adst_t)
        pltpu.sync_copy(ae_hbm, aetab)

        @pl.loop(0, TSLICE // L)
        def _z(i):
            zb[pl.ds(i * L, L)] = jnp.zeros((L,), jnp.float32)

        pltpu.sync_copy(zb, esum_sh.at[pl.ds(sid * TSLICE, TSLICE)])
        plsc.subcore_barrier()

        ii = jnp.arange(L, dtype=jnp.int32)
        rep = lax.shift_right_logical(ii, 2)
        mod = jnp.bitwise_and(ii, 3)
        base_v = ii - mod
        coef = [plsc.load_gather(aetab, [4 * k + mod]) for k in range(DE)]
        ebase0 = wid * (E // NW)

        @pl.loop(0, SUBW)
        def _t(t):
            eb = ebase0 + t * W_EDGES
            pltpu.sync_copy(src_hbm.at[pl.ds(eb, W_EDGES)], srcb)
            pltpu.sync_copy(dst_hbm.at[pl.ds(eb, W_EDGES)], dstb)
            pltpu.sync_copy(ea_hbm.at[pl.ds(4 * eb, 4 * W_EDGES)], eab)

            for g in range(W_EDGES // 4):
                srep = plsc.load_gather(srcb, [g * 4 + rep])
                drep = plsc.load_gather(dstb, [g * 4 + rep])
                sidx = srep * 4 + mod
                didx = drep * 4 + mod
                a1 = plsc.load_gather(asrc_t, [sidx])
                a2 = plsc.load_gather(adst_t, [didx])
                a3 = jnp.zeros((L,), jnp.float32)
                for k in range(DE):
                    ea_k = plsc.load_gather(eab, [g * L + base_v + k])
                    a3 = a3 + ea_k * coef[k]
                e = a1 + a2 + a3
                e = jnp.maximum(e, 0.2 * e)
                wv = jnp.exp(e)
                wb[pl.ds(g * L, L)] = wv
                if g < 16:
                    idxb2[g // 8, pl.ds((g % 8) * L, L)] = didx
                else:
                    idxb1[pl.ds((g - 16) * L, L)] = didx

            pltpu.sync_copy(wb.at[pl.ds(0, 128)],
                            esum_sh.at[idxb2.at[0]], add=True)
            pltpu.sync_copy(wb.at[pl.ds(128, 128)],
                            esum_sh.at[idxb2.at[1]], add=True)
            pltpu.sync_copy(wb.at[pl.ds(256, 64)],
                            esum_sh.at[idxb1], add=True)
            pltpu.sync_copy(wb, w_hbm.at[pl.ds(4 * eb, 4 * W_EDGES)])

        plsc.subcore_barrier()
        pltpu.sync_copy(esum_sh.at[pl.ds(sid * TSLICE, TSLICE)], zb)
        pltpu.sync_copy(zb, part_hbm.at[cid, pl.ds(sid * TSLICE, TSLICE)])

    return kern(asrc_flat, adst_flat, src, dst, ea_flat, ae_flat)


def _message_kernel(inv_flat, src, dst, w_flat, feat_src):
    RZ = 16
    NCH = N // RZ

    @functools.partial(
        pl.kernel,
        mesh=_mesh(),
        out_type=jax.ShapeDtypeStruct((NC, N, HF), jnp.float32),
        scratch_types=[
            pltpu.VMEM((W_EDGES,), jnp.int32),
            pltpu.VMEM((W_EDGES,), jnp.int32),
            pltpu.VMEM((4 * W_EDGES,), jnp.float32),
            pltpu.VMEM((2, 128), jnp.int32),
            pltpu.VMEM((64,), jnp.int32),
            pltpu.VMEM((4 * W_EDGES,), jnp.float32),
            pltpu.VMEM((W_EDGES, HF), jnp.float32),
            pltpu.VMEM((RZ, HF), jnp.float32),
            pltpu.VMEM((TSLICE,), jnp.float32),
            pltpu.VMEM_SHARED((TPAD,), jnp.float32),
            pltpu.VMEM_SHARED((N, HF), jnp.float32),
        ],
        compiler_params=_sc_params(),
    )
    def kern(inv_hbm, src_hbm, dst_hbm, w_hbm, feat_hbm,
             out_hbm,
             srcb, dstb, wb, didxb2, didxb1, invb, rows, zrows, ib,
             inv_sh, acc_sh):
        cid = lax.axis_index("c")
        sid = lax.axis_index("s")
        wid = cid * NS + sid

        pltpu.sync_copy(inv_hbm.at[pl.ds(sid * TSLICE, TSLICE)], ib)
        pltpu.sync_copy(ib, inv_sh.at[pl.ds(sid * TSLICE, TSLICE)])

        @pl.loop(0, RZ)
        def _zr(i):
            for k in range(HF // L):
                zrows[i, pl.ds(k * L, L)] = jnp.zeros((L,), jnp.float32)

        @pl.loop(0, (NCH + NS - 1) // NS)
        def _za(j):
            c = sid + j * NS

            @pl.when(c < NCH)
            def _():
                pltpu.sync_copy(zrows, acc_sh.at[pl.ds(c * RZ, RZ)])

        plsc.subcore_barrier()

        ii = jnp.arange(L, dtype=jnp.int32)
        rep = lax.shift_right_logical(ii, 2)
        mod = jnp.bitwise_and(ii, 3)
        ebase0 = wid * (E // NW)

        @pl.loop(0, SUBW)
        def _t(t):
            eb = ebase0 + t * W_EDGES
            pltpu.sync_copy(src_hbm.at[pl.ds(eb, W_EDGES)], srcb)
            pltpu.sync_copy(dst_hbm.at[pl.ds(eb, W_EDGES)], dstb)
            pltpu.sync_copy(w_hbm.at[pl.ds(4 * eb, 4 * W_EDGES)], wb)
            pltpu.sync_copy(feat_hbm.at[srcb], rows)

            for g in range(W_EDGES // 4):
                drep = plsc.load_gather(dstb, [g * 4 + rep])
                didx = drep * 4 + mod
                if g < 16:
                    didxb2[g // 8, pl.ds((g % 8) * L, L)] = didx
                else:
                    didxb1[pl.ds((g - 16) * L, L)] = didx

            pltpu.sync_copy(inv_sh.at[didxb2.at[0]], invb.at[pl.ds(0, 128)])
            pltpu.sync_copy(inv_sh.at[didxb2.at[1]], invb.at[pl.ds(128, 128)])
            pltpu.sync_copy(inv_sh.at[didxb1], invb.at[pl.ds(256, 64)])

            for g in range(W_EDGES // 4):
                attn = wb[pl.ds(g * L, L)] * invb[pl.ds(g * L, L)]
                for e in range(4):
                    row = g * 4 + e
                    for h in range(H):
                        sp = jnp.broadcast_to(attn[e * 4 + h], (L,))
                        for half in range(2):
                            sl = pl.ds(h * F + half * L, L)
                            rows[row, sl] = rows[row, sl] * sp

            pltpu.sync_copy(rows, acc_sh.at[dstb], add=True)

        plsc.subcore_barrier()

        @pl.loop(0, (NCH + NS - 1) // NS)
        def _wo(j):
            c = sid + j * NS

            @pl.when(c < NCH)
            def _():
                r0 = c * RZ
                pltpu.sync_copy(acc_sh.at[pl.ds(r0, RZ)], zrows)
                pltpu.sync_copy(zrows, out_hbm.at[cid, pl.ds(r0, RZ)])

    return kern(inv_flat, src, dst, w_flat, feat_src)


def kernel(x, edge_index, edge_attr, W_src, W_dst, a_src, a_dst, a_edge):
    src = edge_index[0]
    dst = edge_index[1]
    aW = jnp.concatenate([a_src, a_dst], axis=1)

    feat_src, feat_dst, sa = _node_proj(x, W_src, W_dst, aW)

    asrc_flat = sa[:, :H].reshape(-1)
    adst_flat = sa[:, H:].reshape(-1)
    ea_flat = edge_attr.reshape(-1)
    ae_flat = a_edge.reshape(-1)

    w_flat, esum_part = _attn_kernel(asrc_flat, adst_flat, src, dst,
                                     ea_flat, ae_flat)
    inv_flat = _inv_esum(esum_part)
    out_part = _message_kernel(inv_flat, src, dst, w_flat, feat_src)
    out = _combine(out_part, feat_dst)
    return out.reshape(N, H, F)
